# baseline (speedup 1.0000x reference)
import functools

import jax
import jax.numpy as jnp
from jax import lax
from jax.experimental import pallas as pl
from jax.experimental.pallas import tpu as pltpu
from jax.experimental.pallas import tpu_sc as plsc

_E = 8
_K = 2
_H = 1024
_FF = 2048
_T = 2048
_S = _T * _K
_BT = 128
_NT = _S // _BT
_G = _NT + _E - 1
_LANE = 128


def _router_body(x_ref, wg_ref, wts_ref, ids_ref):
    x = x_ref[...]
    wg = wg_ref[...]
    logits = jnp.dot(x, wg, preferred_element_type=jnp.float32)
    logits = 30.0 * jnp.tanh(logits / 30.0)
    col = lax.broadcasted_iota(jnp.int32, logits.shape, 1)
    real = col < _E
    logits = jnp.where(real, logits, -1e30)
    m = jnp.max(logits, axis=1, keepdims=True)
    ex = jnp.where(real, jnp.exp(logits - m), 0.0)
    p = ex / jnp.sum(ex, axis=1, keepdims=True)
    big = jnp.int32(_LANE)
    m1 = jnp.max(p, axis=1, keepdims=True)
    i1 = jnp.min(jnp.where((p == m1) & real, col, big), axis=1, keepdims=True)
    p2 = jnp.where(col == i1, -1.0, p)
    m2 = jnp.max(p2, axis=1, keepdims=True)
    i2 = jnp.min(jnp.where((p2 == m2) & real, col, big), axis=1, keepdims=True)
    wts_ref[...] = jnp.where(col == 0, m1, jnp.where(col == 1, m2, 0.0))
    ids_ref[...] = jnp.where(col == 0, i1, jnp.where(col == 1, i2, 0))


def _route(x, wg_padded):
    return pl.pallas_call(
        _router_body,
        out_shape=(
            jax.ShapeDtypeStruct((_T, _LANE), jnp.float32),
            jax.ShapeDtypeStruct((_T, _LANE), jnp.int32),
        ),
    )(x, wg_padded)


def _schedule(e_flat, w_flat):
    sort_idx = jnp.argsort(e_flat, stable=True).astype(jnp.int32)
    perm_rows = (sort_idx // _K).astype(jnp.int32)
    w_sorted = jnp.take(w_flat, sort_idx)
    pos = (jnp.zeros((_S,), jnp.int32)
           .at[sort_idx].set(jnp.arange(_S, dtype=jnp.int32)))
    counts = jnp.sum(
        (e_flat[:, None] == jnp.arange(_E, dtype=jnp.int32)[None, :]),
        axis=0, dtype=jnp.int32)
    off = jnp.concatenate(
        [jnp.zeros((1,), jnp.int32), jnp.cumsum(counts).astype(jnp.int32)])
    t_lo = jnp.arange(_NT, dtype=jnp.int32) * _BT
    ov = (off[None, :_E] < (t_lo + _BT)[:, None]) & (off[None, 1:] > t_lo[:, None])
    ov_f = ov.reshape(-1)
    order = jnp.argsort(~ov_f, stable=True).astype(jnp.int32)
    sel = order[:_G]
    valid = jnp.take(ov_f, sel)
    wt = (sel // _E).astype(jnp.int32)
    we = (sel % _E).astype(jnp.int32)
    e_last = jnp.sum((off[1:] <= _S - 1).astype(jnp.int32))
    wt = jnp.where(valid, wt, _NT - 1)
    we = jnp.where(valid, we, e_last)
    lo = jnp.where(valid, jnp.take(off, we), 0)
    hi = jnp.where(valid, jnp.take(off, we + 1), 0)
    prev_wt = jnp.concatenate([jnp.full((1,), -1, jnp.int32), wt[:-1]])
    init = (wt != prev_wt).astype(jnp.int32)
    prev_we = jnp.concatenate([jnp.full((1,), -1, jnp.int32), we[:-1]])
    rstart = (we != prev_we).astype(jnp.int32)
    parity = ((jnp.cumsum(rstart) - 1) % 2).astype(jnp.int32)
    idx = jnp.arange(_G, dtype=jnp.int32)
    cand = jnp.where(rstart == 1, idx, _G)
    rmin = lax.cummin(cand[::-1])[::-1]
    nxt_pos = jnp.concatenate([rmin[1:], jnp.full((1,), _G, jnp.int32)])
    has_nxt = (nxt_pos < _G).astype(jnp.int32)
    nxt_we = jnp.take(we, jnp.minimum(nxt_pos, _G - 1))
    return (sort_idx, perm_rows, w_sorted, pos, wt, we, lo, hi, init,
            rstart, parity, has_nxt, nxt_we)


_SC_CORES = 2
_SC_SUBCORES = 16
_SC_LANES = 16
_NW = _SC_CORES * _SC_SUBCORES


def _sc_gather(x, perm_rows):
    rpw = _S // _NW
    ch = 64
    nch = rpw // ch
    mesh = plsc.VectorSubcoreMesh(core_axis_name="c", subcore_axis_name="s")

    @functools.partial(
        pl.kernel, mesh=mesh,
        out_type=jax.ShapeDtypeStruct((_S, _H), jnp.float32),
        scratch_types=[
            pltpu.VMEM((rpw,), jnp.int32),
            pltpu.VMEM((ch, _H), jnp.float32),
            pltpu.SemaphoreType.DMA,
        ],
    )
    def k(x_hbm, perm_hbm, out_hbm, idx_v, buf, sem):
        wid = lax.axis_index("s") * _SC_CORES + lax.axis_index("c")
        base = wid * rpw
        pltpu.sync_copy(perm_hbm.at[pl.ds(base, rpw)], idx_v)
        for c in range(nch):
            pltpu.async_copy(x_hbm.at[idx_v.at[pl.ds(c * ch, ch)]], buf, sem).wait()
            pltpu.sync_copy(buf, out_hbm.at[pl.ds(base + c * ch, ch)])

    return k(x, perm_rows)


def _sc_combine(ys, pos_a, pos_b):
    tpw = _T // _NW
    ch = 16
    nch = tpw // ch
    nv = _H // _SC_LANES
    mesh = plsc.VectorSubcoreMesh(core_axis_name="c", subcore_axis_name="s")

    @functools.partial(
        pl.kernel, mesh=mesh,
        out_type=jax.ShapeDtypeStruct((_T, _H), jnp.float32),
        scratch_types=[
            pltpu.VMEM((tpw,), jnp.int32),
            pltpu.VMEM((tpw,), jnp.int32),
            pltpu.VMEM((ch, _H), jnp.float32),
            pltpu.VMEM((ch, _H), jnp.float32),
            pltpu.SemaphoreType.DMA,
            pltpu.SemaphoreType.DMA,
        ],
    )
    def k(ys_hbm, pa_hbm, pb_hbm, out_hbm, ia_v, ib_v, abuf, bbuf, sema, semb):
        wid = lax.axis_index("s") * _SC_CORES + lax.axis_index("c")
        base = wid * tpw
        pltpu.sync_copy(pa_hbm.at[pl.ds(base, tpw)], ia_v)
        pltpu.sync_copy(pb_hbm.at[pl.ds(base, tpw)], ib_v)
        for c in range(nch):
            ca = pltpu.async_copy(ys_hbm.at[ia_v.at[pl.ds(c * ch, ch)]], abuf, sema)
            cb = pltpu.async_copy(ys_hbm.at[ib_v.at[pl.ds(c * ch, ch)]], bbuf, semb)
            ca.wait()
            cb.wait()

            def body(r, carry):
                for j in range(nv):
                    sl = pl.ds(j * _SC_LANES, _SC_LANES)
                    abuf[r, sl] = abuf[r, sl] + bbuf[r, sl]
                return carry

            lax.fori_loop(0, ch, body, 0)
            pltpu.sync_copy(abuf, out_hbm.at[pl.ds(base + c * ch, ch)])

    return k(ys, pos_a, pos_b)


def _gelu_exact(x):
    return 0.5 * x * (1.0 + lax.erf(x * (2.0 ** -0.5)))


def _ffn_body(wt_s, we_s, lo_s, hi_s, init_s, rstart_s, parity_s, hasnxt_s, nxtwe_s,
              xs_ref, w1_ref, w3_ref, w2_ref, ws_ref, out_ref,
              w1b, w3b, w2b, sems):
    i = pl.program_id(0)
    p = parity_s[i]

    nsp = 4
    qh = _H // nsp
    qf = _FF // nsp

    def issue(e, slot):
        for q in range(nsp):
            pltpu.make_async_copy(w1_ref.at[e, pl.ds(q * qh, qh)],
                                  w1b.at[slot, pl.ds(q * qh, qh)],
                                  sems.at[slot, 0]).start()
            pltpu.make_async_copy(w3_ref.at[e, pl.ds(q * qh, qh)],
                                  w3b.at[slot, pl.ds(q * qh, qh)],
                                  sems.at[slot, 1]).start()
            pltpu.make_async_copy(w2_ref.at[e, pl.ds(q * qf, qf)],
                                  w2b.at[slot, pl.ds(q * qf, qf)],
                                  sems.at[slot, 2]).start()

    def wait(e, slot):
        pltpu.make_async_copy(w1_ref.at[e], w1b.at[slot], sems.at[slot, 0]).wait()
        pltpu.make_async_copy(w3_ref.at[e], w3b.at[slot], sems.at[slot, 1]).wait()
        pltpu.make_async_copy(w2_ref.at[e], w2b.at[slot], sems.at[slot, 2]).wait()

    @pl.when(i == 0)
    def _():
        issue(we_s[0], 0)

    @pl.when((rstart_s[i] == 1) & (p == 0))
    def _():
        wait(we_s[i], 0)

        @pl.when(hasnxt_s[i] == 1)
        def _():
            issue(nxtwe_s[i], 1)

    @pl.when((rstart_s[i] == 1) & (p == 1))
    def _():
        wait(we_s[i], 1)

        @pl.when(hasnxt_s[i] == 1)
        def _():
            issue(nxtwe_s[i], 0)

    x = xs_ref[...]
    h1 = jnp.dot(x, w1b[p], preferred_element_type=jnp.float32)
    h3 = jnp.dot(x, w3b[p], preferred_element_type=jnp.float32)
    h = _gelu_exact(h1) * h3
    contrib = jnp.dot(h, w2b[p], preferred_element_type=jnp.float32)
    rows = wt_s[i] * _BT + lax.broadcasted_iota(jnp.int32, (_BT, 1), 0)
    mask = (rows >= lo_s[i]) & (rows < hi_s[i])
    coef = jnp.where(mask, ws_ref[0], 0.0)
    contrib = contrib * coef

    @pl.when(init_s[i] == 1)
    def _():
        out_ref[...] = contrib

    @pl.when(init_s[i] == 0)
    def _():
        out_ref[...] += contrib


def _grouped_ffn(xs, w1, w3, w2, ws3, wt, we, lo, hi, init,
                 rstart, parity, has_nxt, nxt_we):
    def im_tile(i, *s):
        return (s[0][i], 0)

    def im_tile3(i, *s):
        return (s[0][i], 0, 0)

    grid_spec = pltpu.PrefetchScalarGridSpec(
        num_scalar_prefetch=9,
        grid=(_G,),
        in_specs=[
            pl.BlockSpec((_BT, _H), im_tile),
            pl.BlockSpec(memory_space=pl.ANY),
            pl.BlockSpec(memory_space=pl.ANY),
            pl.BlockSpec(memory_space=pl.ANY),
            pl.BlockSpec((1, _BT, 1), im_tile3),
        ],
        out_specs=pl.BlockSpec((_BT, _H), im_tile),
        scratch_shapes=[
            pltpu.VMEM((2, _H, _FF), jnp.float32),
            pltpu.VMEM((2, _H, _FF), jnp.float32),
            pltpu.VMEM((2, _FF, _H), jnp.float32),
            pltpu.SemaphoreType.DMA((2, 3)),
        ],
    )
    return pl.pallas_call(
        _ffn_body,
        grid_spec=grid_spec,
        out_shape=jax.ShapeDtypeStruct((_S, _H), jnp.float32),
        compiler_params=pltpu.CompilerParams(
            dimension_semantics=("arbitrary",)),
    )(wt, we, lo, hi, init, rstart, parity, has_nxt, nxt_we,
      xs, w1, w3, w2, ws3)


def kernel(hidden_states, w_gate, w1, w2, w3):
    x = hidden_states.astype(jnp.float32)
    wg_padded = jnp.pad(w_gate.astype(jnp.float32), ((0, 0), (0, _LANE - _E)))
    wts128, ids128 = _route(x, wg_padded)
    w_flat = wts128[:, :_K].reshape(-1)
    e_flat = ids128[:, :_K].reshape(-1)
    (sort_idx, perm_rows, w_sorted, pos, wt, we, lo, hi, init,
     rstart, parity, has_nxt, nxt_we) = _schedule(e_flat, w_flat)
    xs = _sc_gather(x, perm_rows)
    ws3 = w_sorted.reshape(_NT, _BT, 1)
    ys = _grouped_ffn(xs, w1, w3, w2, ws3, wt, we, lo, hi, init,
                      rstart, parity, has_nxt, nxt_we)
    pos_a = pos[0::2]
    pos_b = pos[1::2]
    return _sc_combine(ys, pos_a, pos_b)


# speedup vs baseline: 1.0299x; 1.0299x over previous
import functools

import jax
import jax.numpy as jnp
from jax import lax
from jax.experimental import pallas as pl
from jax.experimental.pallas import tpu as pltpu
from jax.experimental.pallas import tpu_sc as plsc

_E = 8
_K = 2
_H = 1024
_FF = 2048
_T = 2048
_S = _T * _K
_BT = 128
_NT = _S // _BT
_G = _NT + _E - 1
_LANE = 128


def _router_body(x_ref, wg_ref, wts_ref, ids_ref):
    x = x_ref[...]
    wg = wg_ref[...]
    logits = jnp.dot(x, wg, preferred_element_type=jnp.float32)
    logits = 30.0 * jnp.tanh(logits / 30.0)
    col = lax.broadcasted_iota(jnp.int32, logits.shape, 1)
    real = col < _E
    logits = jnp.where(real, logits, -1e30)
    m = jnp.max(logits, axis=1, keepdims=True)
    ex = jnp.where(real, jnp.exp(logits - m), 0.0)
    p = ex / jnp.sum(ex, axis=1, keepdims=True)
    big = jnp.int32(_LANE)
    m1 = jnp.max(p, axis=1, keepdims=True)
    i1 = jnp.min(jnp.where((p == m1) & real, col, big), axis=1, keepdims=True)
    p2 = jnp.where(col == i1, -1.0, p)
    m2 = jnp.max(p2, axis=1, keepdims=True)
    i2 = jnp.min(jnp.where((p2 == m2) & real, col, big), axis=1, keepdims=True)
    wts_ref[...] = jnp.where(col == 0, m1, jnp.where(col == 1, m2, 0.0))
    ids_ref[...] = jnp.where(col == 0, i1, jnp.where(col == 1, i2, 0))


def _route(x, wg_padded):
    return pl.pallas_call(
        _router_body,
        out_shape=(
            jax.ShapeDtypeStruct((_T, _LANE), jnp.float32),
            jax.ShapeDtypeStruct((_T, _LANE), jnp.int32),
        ),
    )(x, wg_padded)


def _schedule(e_flat, w_flat):
    sort_idx = jnp.argsort(e_flat, stable=True).astype(jnp.int32)
    perm_rows = (sort_idx // _K).astype(jnp.int32)
    w_sorted = jnp.take(w_flat, sort_idx)
    pos = (jnp.zeros((_S,), jnp.int32)
           .at[sort_idx].set(jnp.arange(_S, dtype=jnp.int32)))
    counts = jnp.sum(
        (e_flat[:, None] == jnp.arange(_E, dtype=jnp.int32)[None, :]),
        axis=0, dtype=jnp.int32)
    off = jnp.concatenate(
        [jnp.zeros((1,), jnp.int32), jnp.cumsum(counts).astype(jnp.int32)])
    t_lo = jnp.arange(_NT, dtype=jnp.int32) * _BT
    ov = (off[None, :_E] < (t_lo + _BT)[:, None]) & (off[None, 1:] > t_lo[:, None])
    ov_f = ov.reshape(-1)
    order = jnp.argsort(~ov_f, stable=True).astype(jnp.int32)
    sel = order[:_G]
    valid = jnp.take(ov_f, sel)
    wt = (sel // _E).astype(jnp.int32)
    we = (sel % _E).astype(jnp.int32)
    e_last = jnp.sum((off[1:] <= _S - 1).astype(jnp.int32))
    wt = jnp.where(valid, wt, _NT - 1)
    we = jnp.where(valid, we, e_last)
    lo = jnp.where(valid, jnp.take(off, we), 0)
    hi = jnp.where(valid, jnp.take(off, we + 1), 0)
    prev_wt = jnp.concatenate([jnp.full((1,), -1, jnp.int32), wt[:-1]])
    init = (wt != prev_wt).astype(jnp.int32)
    prev_we = jnp.concatenate([jnp.full((1,), -1, jnp.int32), we[:-1]])
    rstart = (we != prev_we).astype(jnp.int32)
    parity = ((jnp.cumsum(rstart) - 1) % 2).astype(jnp.int32)
    idx = jnp.arange(_G, dtype=jnp.int32)
    cand = jnp.where(rstart == 1, idx, _G)
    rmin = lax.cummin(cand[::-1])[::-1]
    nxt_pos = jnp.concatenate([rmin[1:], jnp.full((1,), _G, jnp.int32)])
    has_nxt = (nxt_pos < _G).astype(jnp.int32)
    nxt_we = jnp.take(we, jnp.minimum(nxt_pos, _G - 1))
    return (sort_idx, perm_rows, w_sorted, pos, wt, we, lo, hi, init,
            rstart, parity, has_nxt, nxt_we)


_SC_CORES = 2
_SC_SUBCORES = 16
_SC_LANES = 16
_NW = _SC_CORES * _SC_SUBCORES


def _sc_gather(x, perm_rows):
    rpw = _S // _NW
    ch = 32
    nch = rpw // ch
    mesh = plsc.VectorSubcoreMesh(core_axis_name="c", subcore_axis_name="s")

    @functools.partial(
        pl.kernel, mesh=mesh,
        out_type=jax.ShapeDtypeStruct((_S, _H), jnp.float32),
        scratch_types=[
            pltpu.VMEM((rpw,), jnp.int32),
            pltpu.VMEM((ch, _H), jnp.float32),
            pltpu.VMEM((ch, _H), jnp.float32),
            pltpu.SemaphoreType.DMA,
            pltpu.SemaphoreType.DMA,
        ],
    )
    def k(x_hbm, perm_hbm, out_hbm, idx_v, buf0, buf1, sem0, sem1):
        wid = lax.axis_index("s") * _SC_CORES + lax.axis_index("c")
        base = wid * rpw
        pltpu.sync_copy(perm_hbm.at[pl.ds(base, rpw)], idx_v)
        bufs = (buf0, buf1)
        sems = (sem0, sem1)
        cps = []
        for c in range(nch):
            cps.append(pltpu.async_copy(
                x_hbm.at[idx_v.at[pl.ds(c * ch, ch)]], bufs[c % 2], sems[c % 2]))
        for c in range(nch):
            cps[c].wait()
            pltpu.sync_copy(bufs[c % 2], out_hbm.at[pl.ds(base + c * ch, ch)])

    return k(x, perm_rows)


def _sc_combine(ys, pos_a, pos_b):
    tpw = _T // _NW
    ch = 16
    nch = tpw // ch
    nv = _H // _SC_LANES
    mesh = plsc.VectorSubcoreMesh(core_axis_name="c", subcore_axis_name="s")

    @functools.partial(
        pl.kernel, mesh=mesh,
        out_type=jax.ShapeDtypeStruct((_T, _H), jnp.float32),
        scratch_types=[
            pltpu.VMEM((tpw,), jnp.int32),
            pltpu.VMEM((tpw,), jnp.int32),
            pltpu.VMEM((ch, _H), jnp.float32),
            pltpu.VMEM((ch, _H), jnp.float32),
            pltpu.VMEM((ch, _H), jnp.float32),
            pltpu.VMEM((ch, _H), jnp.float32),
            pltpu.SemaphoreType.DMA,
            pltpu.SemaphoreType.DMA,
            pltpu.SemaphoreType.DMA,
        ],
    )
    def k(ys_hbm, pa_hbm, pb_hbm, out_hbm, ia_v, ib_v, abuf0, abuf1,
          bbuf0, bbuf1, sema, semb, semw):
        wid = lax.axis_index("s") * _SC_CORES + lax.axis_index("c")
        base = wid * tpw
        pltpu.sync_copy(pa_hbm.at[pl.ds(base, tpw)], ia_v)
        pltpu.sync_copy(pb_hbm.at[pl.ds(base, tpw)], ib_v)

        abufs = (abuf0, abuf1)
        bbufs = (bbuf0, bbuf1)

        def gather(c):
            sl = c % 2
            return (
                pltpu.async_copy(ys_hbm.at[ia_v.at[pl.ds(c * ch, ch)]],
                                 abufs[sl], sema),
                pltpu.async_copy(ys_hbm.at[ib_v.at[pl.ds(c * ch, ch)]],
                                 bbufs[sl], semb),
            )

        cps = [gather(0), gather(1)]
        for c in range(nch):
            ca, cb = cps[c % 2]
            ca.wait()
            cb.wait()
            ab = abufs[c % 2]
            bb = bbufs[c % 2]

            def body(r, carry):
                for j in range(nv):
                    ds = pl.ds(j * _SC_LANES, _SC_LANES)
                    ab[r, ds] = ab[r, ds] + bb[r, ds]
                return carry

            lax.fori_loop(0, ch, body, 0)
            wr = pltpu.make_async_copy(ab,
                                       out_hbm.at[pl.ds(base + c * ch, ch)], semw)
            wr.start()
            wr.wait()
            if c + 2 < nch:
                cps[c % 2] = gather(c + 2)

    return k(ys, pos_a, pos_b)


def _gelu_exact(x):
    return 0.5 * x * (1.0 + lax.erf(x * (2.0 ** -0.5)))


def _ffn_body(wt_s, we_s, lo_s, hi_s, init_s, rstart_s, parity_s, hasnxt_s, nxtwe_s,
              xs_ref, w1_ref, w3_ref, w2_ref, ws_ref, out_ref,
              w1b, w3b, w2b, sems):
    i = pl.program_id(0)
    p = parity_s[i]

    nsp = 4
    qh = _H // nsp
    qf = _FF // nsp

    def issue(e, slot):
        for q in range(nsp):
            pltpu.make_async_copy(w1_ref.at[e, pl.ds(q * qh, qh)],
                                  w1b.at[slot, pl.ds(q * qh, qh)],
                                  sems.at[slot, 0]).start()
            pltpu.make_async_copy(w3_ref.at[e, pl.ds(q * qh, qh)],
                                  w3b.at[slot, pl.ds(q * qh, qh)],
                                  sems.at[slot, 1]).start()
            pltpu.make_async_copy(w2_ref.at[e, pl.ds(q * qf, qf)],
                                  w2b.at[slot, pl.ds(q * qf, qf)],
                                  sems.at[slot, 2]).start()

    def wait(e, slot):
        pltpu.make_async_copy(w1_ref.at[e], w1b.at[slot], sems.at[slot, 0]).wait()
        pltpu.make_async_copy(w3_ref.at[e], w3b.at[slot], sems.at[slot, 1]).wait()
        pltpu.make_async_copy(w2_ref.at[e], w2b.at[slot], sems.at[slot, 2]).wait()

    @pl.when(i == 0)
    def _():
        issue(we_s[0], 0)

    @pl.when((rstart_s[i] == 1) & (p == 0))
    def _():
        wait(we_s[i], 0)

        @pl.when(hasnxt_s[i] == 1)
        def _():
            issue(nxtwe_s[i], 1)

    @pl.when((rstart_s[i] == 1) & (p == 1))
    def _():
        wait(we_s[i], 1)

        @pl.when(hasnxt_s[i] == 1)
        def _():
            issue(nxtwe_s[i], 0)

    x = xs_ref[...]
    h1 = jnp.dot(x, w1b[p], preferred_element_type=jnp.float32)
    h3 = jnp.dot(x, w3b[p], preferred_element_type=jnp.float32)
    h = _gelu_exact(h1) * h3
    contrib = jnp.dot(h, w2b[p], preferred_element_type=jnp.float32)
    rows = wt_s[i] * _BT + lax.broadcasted_iota(jnp.int32, (_BT, 1), 0)
    mask = (rows >= lo_s[i]) & (rows < hi_s[i])
    coef = jnp.where(mask, ws_ref[0], 0.0)
    contrib = contrib * coef

    @pl.when(init_s[i] == 1)
    def _():
        out_ref[...] = contrib

    @pl.when(init_s[i] == 0)
    def _():
        out_ref[...] += contrib


def _grouped_ffn(xs, w1, w3, w2, ws3, wt, we, lo, hi, init,
                 rstart, parity, has_nxt, nxt_we):
    def im_tile(i, *s):
        return (s[0][i], 0)

    def im_tile3(i, *s):
        return (s[0][i], 0, 0)

    grid_spec = pltpu.PrefetchScalarGridSpec(
        num_scalar_prefetch=9,
        grid=(_G,),
        in_specs=[
            pl.BlockSpec((_BT, _H), im_tile),
            pl.BlockSpec(memory_space=pl.ANY),
            pl.BlockSpec(memory_space=pl.ANY),
            pl.BlockSpec(memory_space=pl.ANY),
            pl.BlockSpec((1, _BT, 1), im_tile3),
        ],
        out_specs=pl.BlockSpec((_BT, _H), im_tile),
        scratch_shapes=[
            pltpu.VMEM((2, _H, _FF), jnp.float32),
            pltpu.VMEM((2, _H, _FF), jnp.float32),
            pltpu.VMEM((2, _FF, _H), jnp.float32),
            pltpu.SemaphoreType.DMA((2, 3)),
        ],
    )
    return pl.pallas_call(
        _ffn_body,
        grid_spec=grid_spec,
        out_shape=jax.ShapeDtypeStruct((_S, _H), jnp.float32),
        compiler_params=pltpu.CompilerParams(
            dimension_semantics=("arbitrary",)),
    )(wt, we, lo, hi, init, rstart, parity, has_nxt, nxt_we,
      xs, w1, w3, w2, ws3)


def kernel(hidden_states, w_gate, w1, w2, w3):
    x = hidden_states.astype(jnp.float32)
    wg_padded = jnp.pad(w_gate.astype(jnp.float32), ((0, 0), (0, _LANE - _E)))
    wts128, ids128 = _route(x, wg_padded)
    w_flat = wts128[:, :_K].reshape(-1)
    e_flat = ids128[:, :_K].reshape(-1)
    (sort_idx, perm_rows, w_sorted, pos, wt, we, lo, hi, init,
     rstart, parity, has_nxt, nxt_we) = _schedule(e_flat, w_flat)
    xs = _sc_gather(x, perm_rows)
    ws3 = w_sorted.reshape(_NT, _BT, 1)
    ys = _grouped_ffn(xs, w1, w3, w2, ws3, wt, we, lo, hi, init,
                      rstart, parity, has_nxt, nxt_we)
    pos_a = pos[0::2]
    pos_b = pos[1::2]
    return _sc_combine(ys, pos_a, pos_b)
